# baseline (speedup 1.0000x reference)
import functools

import jax
import jax.numpy as jnp
from jax import lax
from jax.experimental import pallas as pl
from jax.experimental.pallas import tpu as pltpu
from jax.experimental.pallas import tpu_sc as plsc

_NC = 2
_NS = 16
_NW = _NC * _NS
_L = 16

_CHUNK_B = 16


def _stab_body(f_ref, w_ref, b_ref, v_ref, o_ref, fb_ref):
    f = f_ref[...]
    x = jnp.dot(f, w_ref[...], preferred_element_type=jnp.float32)
    t = jnp.tanh(x + b_ref[...])
    o_ref[...] = jnp.dot(t, v_ref[...], preferred_element_type=jnp.float32)
    fb_ref[...] = f.astype(jnp.bfloat16)


def _score_table(features, W_att, b_att, v_ctx):
    n, d = features.shape
    att = W_att.shape[1]
    blk = 5000
    grid = n // blk
    out, fbf = pl.pallas_call(
        _stab_body,
        grid=(grid,),
        in_specs=[
            pl.BlockSpec((blk, d), lambda i: (i, 0)),
            pl.BlockSpec((d, att), lambda i: (0, 0)),
            pl.BlockSpec((1, att), lambda i: (0, 0)),
            pl.BlockSpec((att, 1), lambda i: (0, 0)),
        ],
        out_specs=[
            pl.BlockSpec((blk, 1), lambda i: (i, 0)),
            pl.BlockSpec((blk, d), lambda i: (i, 0)),
        ],
        out_shape=[
            jax.ShapeDtypeStruct((n, 1), jnp.float32),
            jax.ShapeDtypeStruct((n, d), jnp.bfloat16),
        ],
    )(features, W_att, b_att.reshape(1, att), v_ctx)
    return out.reshape(n), fbf


def _make_sc_agg(n_nodes, d_feat, s_nbr, ct_total):
    S = s_nbr
    RPC = _CHUNK_B * S
    q, r = divmod(ct_total, _NW)
    CPW = q + 1
    NBUF = 3
    HB = _CHUNK_B // 2
    half = HB * S
    mesh = plsc.VectorSubcoreMesh(
        core_axis_name="c", subcore_axis_name="s",
        num_cores=_NC, num_subcores=_NS)

    @functools.partial(
        pl.kernel,
        out_type=jax.ShapeDtypeStruct((ct_total * _CHUNK_B, d_feat),
                                      jnp.float32),
        mesh=mesh,
        compiler_params=pltpu.CompilerParams(
            use_tc_tiling_on_sc=False, needs_layout_passes=False),
        scratch_types=[
            pltpu.VMEM((CPW * 2, half), jnp.int32),
            pltpu.VMEM((n_nodes,), jnp.float32),
            pltpu.VMEM((NBUF, RPC, d_feat), jnp.bfloat16),
            pltpu.VMEM((S, _L), jnp.float32),
            pltpu.VMEM((_CHUNK_B, d_feat), jnp.float32),
            pltpu.VMEM((_L,), jnp.float32),
            pltpu.SemaphoreType.DMA,
            pltpu.SemaphoreType.DMA,
            pltpu.SemaphoreType.DMA,
            pltpu.SemaphoreType.DMA,
        ],
    )
    def sc_agg(feat_hbm, nidx_hbm, stab_hbm, scale_hbm, out_hbm,
               idx_v, stab_v, rows_v, w_v, ob_v, sc_v,
               sem_r0, sem_r1, sem_r2, sem_o):
        wid = lax.axis_index("s") * _NC + lax.axis_index("c")
        start = q * wid + jnp.minimum(wid, r)

        pltpu.sync_copy(stab_hbm, stab_v)

        @pl.when(start + CPW <= ct_total)
        def _():
            pltpu.sync_copy(
                nidx_hbm.at[pl.ds(start * 2, CPW * 2)], idx_v)

        @pl.when(start + CPW > ct_total)
        def _():
            pltpu.sync_copy(
                nidx_hbm.at[pl.ds(start * 2, (CPW - 1) * 2)],
                idx_v.at[pl.ds(0, (CPW - 1) * 2)])
            zero = jnp.zeros((_L,), jnp.int32)
            for row in range(CPW * 2 - 2, CPW * 2):
                for j in range(half // _L):
                    idx_v[row, pl.ds(j * _L, _L)] = zero

        pltpu.sync_copy(scale_hbm, sc_v)
        sv = sc_v[...]
        sem_r = (sem_r0, sem_r1, sem_r2)

        def issue(k, p):
            for h in range(2):
                pltpu.async_copy(
                    feat_hbm.at[idx_v.at[k * 2 + h]],
                    rows_v.at[p, pl.ds(h * half, half)], sem_r[p])

        def chunk_indices(k):
            lane_flat = lax.iota(jnp.int32, _L) * S
            out = []
            for s in range(S):
                t = lane_flat + s
                hi = (t >= half).astype(jnp.int32)
                out.append(plsc.load_gather(
                    idx_v, [k * 2 + hi, t - hi * half]))
            return out

        def process(k, p):
            scores = [plsc.load_gather(stab_v, [iv])
                      for iv in chunk_indices(k)]
            m = scores[0]
            for s in range(1, S):
                m = jnp.maximum(m, scores[s])
            exps = [jnp.exp(x - m) for x in scores]
            tot = exps[0]
            for s in range(1, S):
                tot = tot + exps[s]
            wfac = sv / tot
            for s in range(S):
                w_v[s] = exps[s] * wfac

            @pl.when(k >= 1)
            def _():
                pltpu.make_async_copy(
                    out_hbm.at[pl.ds(0, _CHUNK_B)], ob_v,
                    sem_o).wait()

            pltpu.make_async_copy(
                feat_hbm.at[pl.ds(0, RPC)], rows_v.at[p], sem_r[p]).wait()

            @plsc.parallel_loop(0, _CHUNK_B)
            def b_body(b):
                    bidx = jnp.full((_L,), b, jnp.int32)
                    wb = [
                        plsc.load_gather(
                            w_v, [jnp.full((_L,), s, jnp.int32), bidx])
                        for s in range(S)
                    ]
                    def tree(terms):
                        while len(terms) > 1:
                            nxt = [terms[j] + terms[j + 1]
                                   for j in range(0, len(terms) - 1, 2)]
                            if len(terms) % 2:
                                nxt.append(terms[-1])
                            terms = nxt
                        return terms[0]

                    lane2 = lax.iota(jnp.int32, _L) * 2
                    brow = jnp.full((_L,), b, jnp.int32)
                    for kk in range(d_feat // (2 * _L)):
                        base = kk * 2 * _L
                        ta, tb = [], []
                        for s in range(S):
                            rv = rows_v[p, b * S + s, pl.ds(base, 2 * _L)]
                            ea, eb = plsc.unpack(
                                rv, format=plsc.PackFormat.INTERLEAVED,
                                preferred_element_type=jnp.float32)
                            ta.append(wb[s] * ea)
                            tb.append(wb[s] * eb)
                        va = jnp.maximum(tree(ta), 0.0)
                        vb = jnp.maximum(tree(tb), 0.0)
                        plsc.store_scatter(ob_v, [brow, base + lane2], va)
                        plsc.store_scatter(ob_v, [brow, base + lane2 + 1], vb)

            @pl.when(start + k < ct_total)
            def _():
                row0 = (start + k) * _CHUNK_B
                pltpu.async_copy(
                    ob_v, out_hbm.at[pl.ds(row0, _CHUNK_B)], sem_o)

        issue(0, 0)
        issue(1, 1)

        def trip_body(i, _):
            for p in range(NBUF):
                k = NBUF * i + p

                @pl.when(k < CPW)
                def _():
                    @pl.when(k + 2 < CPW)
                    def _():
                        issue(k + 2, (p + 2) % NBUF)

                    process(k, p)

            return _

        lax.fori_loop(0, (CPW + NBUF - 1) // NBUF, trip_body, None)

        @pl.when(start + CPW - 1 < ct_total)
        def _():
            pltpu.make_async_copy(
                out_hbm.at[pl.ds(0, _CHUNK_B)], ob_v, sem_o).wait()

    return sc_agg


def kernel(features, nodes, neigh_idx, W_att, b_att, v_ctx, num_sample):
    del nodes
    n_nodes, d_feat = features.shape
    b_sz, s_nbr = neigh_idx.shape
    assert b_sz % _CHUNK_B == 0
    ct_total = b_sz // _CHUNK_B

    sc_agg = _make_sc_agg(n_nodes, d_feat, s_nbr, ct_total)

    stab, fbf = _score_table(features, W_att, b_att, v_ctx)
    scale = jnp.full((_L,), 1.0, jnp.float32) / num_sample

    half_rows = neigh_idx.reshape(2 * ct_total, (_CHUNK_B // 2) * s_nbr)
    return sc_agg(fbf, half_rows, stab, scale)


# speedup vs baseline: 1.1819x; 1.1819x over previous
import functools

import jax
import jax.numpy as jnp
from jax import lax
from jax.experimental import pallas as pl
from jax.experimental.pallas import tpu as pltpu
from jax.experimental.pallas import tpu_sc as plsc

_NC = 2
_NS = 16
_NW = _NC * _NS
_L = 16

_CHUNK_B = 16


def _stab_body(f_ref, w_ref, b_ref, v_ref, o_ref):
    x = jnp.dot(f_ref[...], w_ref[...], preferred_element_type=jnp.float32)
    t = jnp.tanh(x + b_ref[...])
    o_ref[...] = jnp.dot(t, v_ref[...], preferred_element_type=jnp.float32)


def _score_table(features, W_att, b_att, v_ctx):
    n, d = features.shape
    att = W_att.shape[1]
    blk = 5000
    grid = n // blk
    out = pl.pallas_call(
        _stab_body,
        grid=(grid,),
        in_specs=[
            pl.BlockSpec((blk, d), lambda i: (i, 0)),
            pl.BlockSpec((d, att), lambda i: (0, 0)),
            pl.BlockSpec((1, att), lambda i: (0, 0)),
            pl.BlockSpec((att, 1), lambda i: (0, 0)),
        ],
        out_specs=pl.BlockSpec((blk, 1), lambda i: (i, 0)),
        out_shape=jax.ShapeDtypeStruct((n, 1), jnp.float32),
    )(features, W_att, b_att.reshape(1, att), v_ctx)
    return out.reshape(n)


def _make_sc_agg(n_nodes, d_feat, s_nbr, ct_total):
    S = s_nbr
    RPC = _CHUNK_B * S
    q, r = divmod(ct_total, _NW)
    CPW = q + 1
    NBUF = 3
    HB = _CHUNK_B // 2
    half = HB * S
    mesh = plsc.VectorSubcoreMesh(
        core_axis_name="c", subcore_axis_name="s",
        num_cores=_NC, num_subcores=_NS)

    @functools.partial(
        pl.kernel,
        out_type=jax.ShapeDtypeStruct((ct_total * _CHUNK_B, d_feat),
                                      jnp.float32),
        mesh=mesh,
        compiler_params=pltpu.CompilerParams(
            use_tc_tiling_on_sc=False, needs_layout_passes=False),
        scratch_types=[
            pltpu.VMEM((CPW * 2, half), jnp.int32),
            pltpu.VMEM((n_nodes,), jnp.float32),
            pltpu.VMEM((NBUF, RPC, d_feat), jnp.float32),
            pltpu.VMEM((S, _L), jnp.float32),
            pltpu.VMEM((_CHUNK_B, d_feat), jnp.float32),
            pltpu.VMEM((_L,), jnp.float32),
            pltpu.SemaphoreType.DMA,
            pltpu.SemaphoreType.DMA,
            pltpu.SemaphoreType.DMA,
            pltpu.SemaphoreType.DMA,
        ],
    )
    def sc_agg(feat_hbm, nidx_hbm, stab_hbm, scale_hbm, out_hbm,
               idx_v, stab_v, rows_v, w_v, ob_v, sc_v,
               sem_r0, sem_r1, sem_r2, sem_o):
        wid = lax.axis_index("s") * _NC + lax.axis_index("c")
        start = q * wid + jnp.minimum(wid, r)

        pltpu.sync_copy(stab_hbm, stab_v)

        @pl.when(start + CPW <= ct_total)
        def _():
            pltpu.sync_copy(
                nidx_hbm.at[pl.ds(start * 2, CPW * 2)], idx_v)

        @pl.when(start + CPW > ct_total)
        def _():
            pltpu.sync_copy(
                nidx_hbm.at[pl.ds(start * 2, (CPW - 1) * 2)],
                idx_v.at[pl.ds(0, (CPW - 1) * 2)])
            zero = jnp.zeros((_L,), jnp.int32)
            for row in range(CPW * 2 - 2, CPW * 2):
                for j in range(half // _L):
                    idx_v[row, pl.ds(j * _L, _L)] = zero

        pltpu.sync_copy(scale_hbm, sc_v)
        sv = sc_v[...]
        sem_r = (sem_r0, sem_r1, sem_r2)

        def issue(k, p):
            for h in range(2):
                pltpu.async_copy(
                    feat_hbm.at[idx_v.at[k * 2 + h]],
                    rows_v.at[p, pl.ds(h * half, half)], sem_r[p])

        def chunk_indices(k):
            lane_flat = lax.iota(jnp.int32, _L) * S
            out = []
            for s in range(S):
                t = lane_flat + s
                hi = (t >= half).astype(jnp.int32)
                out.append(plsc.load_gather(
                    idx_v, [k * 2 + hi, t - hi * half]))
            return out

        def process(k, p):
            scores = [plsc.load_gather(stab_v, [iv])
                      for iv in chunk_indices(k)]
            m = scores[0]
            for s in range(1, S):
                m = jnp.maximum(m, scores[s])
            exps = [jnp.exp(x - m) for x in scores]
            tot = exps[0]
            for s in range(1, S):
                tot = tot + exps[s]
            wfac = sv / tot
            for s in range(S):
                w_v[s] = exps[s] * wfac

            @pl.when(k >= 1)
            def _():
                pltpu.make_async_copy(
                    out_hbm.at[pl.ds(0, _CHUNK_B)], ob_v,
                    sem_o).wait()

            pltpu.make_async_copy(
                feat_hbm.at[pl.ds(0, RPC)], rows_v.at[p], sem_r[p]).wait()

            @plsc.parallel_loop(0, _CHUNK_B)
            def b_body(b):
                    bidx = jnp.full((_L,), b, jnp.int32)
                    wb = [
                        plsc.load_gather(
                            w_v, [jnp.full((_L,), s, jnp.int32), bidx])
                        for s in range(S)
                    ]
                    for kk in range(d_feat // _L):
                        ks = pl.ds(kk * _L, _L)
                        terms = [wb[s] * rows_v[p, b * S + s, ks]
                                 for s in range(S)]
                        while len(terms) > 1:
                            nxt = [terms[j] + terms[j + 1]
                                   for j in range(0, len(terms) - 1, 2)]
                            if len(terms) % 2:
                                nxt.append(terms[-1])
                            terms = nxt
                        ob_v[b, ks] = jnp.maximum(terms[0], 0.0)

            @pl.when(start + k < ct_total)
            def _():
                row0 = (start + k) * _CHUNK_B
                pltpu.async_copy(
                    ob_v, out_hbm.at[pl.ds(row0, _CHUNK_B)], sem_o)

        issue(0, 0)
        issue(1, 1)

        def trip_body(i, _):
            for p in range(NBUF):
                k = NBUF * i + p

                @pl.when(k < CPW)
                def _():
                    @pl.when(k + 2 < CPW)
                    def _():
                        issue(k + 2, (p + 2) % NBUF)

                    process(k, p)

            return _

        lax.fori_loop(0, (CPW + NBUF - 1) // NBUF, trip_body, None)

        @pl.when(start + CPW - 1 < ct_total)
        def _():
            pltpu.make_async_copy(
                out_hbm.at[pl.ds(0, _CHUNK_B)], ob_v, sem_o).wait()

    return sc_agg


def kernel(features, nodes, neigh_idx, W_att, b_att, v_ctx, num_sample):
    del nodes
    n_nodes, d_feat = features.shape
    b_sz, s_nbr = neigh_idx.shape
    assert b_sz % _CHUNK_B == 0
    ct_total = b_sz // _CHUNK_B

    sc_agg = _make_sc_agg(n_nodes, d_feat, s_nbr, ct_total)

    stab = _score_table(features, W_att, b_att, v_ctx)
    scale = jnp.full((_L,), 1.0, jnp.float32) / num_sample

    half_rows = neigh_idx.reshape(2 * ct_total, (_CHUNK_B // 2) * s_nbr)
    return sc_agg(features, half_rows, stab, scale)
